# baseline (speedup 1.0000x reference)
import jax
import jax.numpy as jnp
from jax.experimental import pallas as pl

_B, _T, _N, _D = 16, 64, 256, 96
_HID = 4
_KF, _KT = 16, 64


def _gelu(x):
    return 0.5 * x * (1.0 + jax.lax.erf(x * (2.0 ** -0.5)))


def _topk_idx_rows(soft, k, iota_cols, n):
    rows = soft.shape[0]
    iota_k = jax.lax.broadcasted_iota(jnp.int32, (rows, k), 1)

    def step(i, c):
        work, acc = c
        mx = jnp.max(work, axis=-1, keepdims=True)
        sel = jnp.min(jnp.where(work == mx, iota_cols, n), axis=-1, keepdims=True)
        acc = jnp.where(iota_k == i, sel, acc)
        work = jnp.where(iota_cols == sel, -jnp.inf, work)
        return work, acc

    _, idx = jax.lax.fori_loop(
        0, k, step, (soft, jnp.zeros((rows, k), jnp.int32)))
    return idx


def _body(x_ref, mk_ref, fg_ref, fbeta_ref, fw1_ref, fb1_ref, fw2_ref, fb2_ref,
          tg_ref, tbeta_ref, tw1_ref, tb1_ref, tw2_ref, tb2_ref,
          tm_ref, fm_ref, fi_ref, ti_ref):
    xb = x_ref[0]
    mk = mk_ref[0]

    denom = jnp.clip(jnp.sum(mk, axis=1, keepdims=True), 1e-6, None)
    fr = jnp.sum(xb * mk[:, :, None], axis=1) / denom

    m = jnp.mean(fr, axis=-1, keepdims=True)
    v = jnp.mean((fr - m) ** 2, axis=-1, keepdims=True)
    h = (fr - m) / jnp.sqrt(v + 1e-5) * fg_ref[0] + fbeta_ref[0]
    h = jnp.dot(h, fw1_ref[...], preferred_element_type=jnp.float32) + fb1_ref[0]
    h = _gelu(h)
    fl = jnp.dot(h, fw2_ref[...], preferred_element_type=jnp.float32) + fb2_ref[0]

    fe = jnp.exp(fl - jnp.max(fl, axis=0, keepdims=True))
    fs = fe / jnp.sum(fe, axis=0, keepdims=True)

    iota_t = jax.lax.broadcasted_iota(jnp.int32, (_T, 1), 0)
    iota_kf = jax.lax.broadcasted_iota(jnp.int32, (_KF, 1), 0)

    def fstep(i, c):
        work, acc = c
        mx = jnp.max(work, axis=0, keepdims=True)
        sel = jnp.min(jnp.where(work == mx, iota_t, _T), axis=0, keepdims=True)
        acc = jnp.where(iota_kf == i, sel, acc)
        work = jnp.where(iota_t == sel, -jnp.inf, work)
        return work, acc

    _, fidx = jax.lax.fori_loop(
        0, _KF, fstep, (fs, jnp.zeros((_KF, 1), jnp.int32)))

    xt = xb.reshape(_T * _N, _D)
    tm_ = jnp.mean(xt, axis=-1, keepdims=True)
    tv = jnp.mean((xt - tm_) ** 2, axis=-1, keepdims=True)
    t = (xt - tm_) / jnp.sqrt(tv + 1e-5) * tg_ref[0] + tbeta_ref[0]
    t = jnp.dot(t, tw1_ref[...], preferred_element_type=jnp.float32) + tb1_ref[0]
    t = _gelu(t)
    tl = jnp.dot(t, tw2_ref[...], preferred_element_type=jnp.float32) + tb2_ref[0]
    tl = tl.reshape(_T, _N) + jnp.log(jnp.clip(mk, 1e-9, None))

    te = jnp.exp(tl - jnp.max(tl, axis=-1, keepdims=True))
    ts = te / jnp.sum(te, axis=-1, keepdims=True)

    iota_n = jax.lax.broadcasted_iota(jnp.int32, (_T, _N), 1)
    tidx = _topk_idx_rows(ts, _KT, iota_n, _N)

    tm_ref[0] = ts * fs
    fm_ref[0] = fs
    fi_ref[0] = fidx
    ti_ref[0] = tidx


def kernel(x, mask, fm_ln_g, fm_ln_b, fm_w1, fm_b1, fm_w2, fm_b2,
           tk_ln_g, tk_ln_b, tk_w1, tk_b1, tk_w2, tk_b2):
    row = lambda w: w.reshape(1, -1)
    grid = (_B,)
    bcast = lambda shape: pl.BlockSpec(shape, lambda b: (0,) * len(shape))
    out = pl.pallas_call(
        _body,
        grid=grid,
        in_specs=[
            pl.BlockSpec((1, _T, _N, _D), lambda b: (b, 0, 0, 0)),
            pl.BlockSpec((1, _T, _N), lambda b: (b, 0, 0)),
            bcast((1, _D)), bcast((1, _D)),
            bcast((_D, _HID * _D)), bcast((1, _HID * _D)),
            bcast((_HID * _D, 1)), bcast((1, 1)),
            bcast((1, _D)), bcast((1, _D)),
            bcast((_D, _D // 2)), bcast((1, _D // 2)),
            bcast((_D // 2, 1)), bcast((1, 1)),
        ],
        out_specs=[
            pl.BlockSpec((1, _T, _N), lambda b: (b, 0, 0)),
            pl.BlockSpec((1, _T, 1), lambda b: (b, 0, 0)),
            pl.BlockSpec((1, _KF, 1), lambda b: (b, 0, 0)),
            pl.BlockSpec((1, _T, _KT), lambda b: (b, 0, 0)),
        ],
        out_shape=[
            jax.ShapeDtypeStruct((_B, _T, _N), jnp.float32),
            jax.ShapeDtypeStruct((_B, _T, 1), jnp.float32),
            jax.ShapeDtypeStruct((_B, _KF, 1), jnp.int32),
            jax.ShapeDtypeStruct((_B, _T, _KT), jnp.int32),
        ],
    )(x, mask,
      row(fm_ln_g), row(fm_ln_b), fm_w1, row(fm_b1), fm_w2, row(fm_b2),
      row(tk_ln_g), row(tk_ln_b), tk_w1, row(tk_b1), tk_w2, row(tk_b2))
    token_mask, frame_mask, frame_idx, token_idx = out
    return (token_mask, frame_mask.reshape(_B, _T),
            frame_idx.reshape(_B, _KF), token_idx)
